# baseline (speedup 1.0000x reference)
import functools

import jax
import jax.numpy as jnp
from jax import lax
from jax.experimental import pallas as pl
from jax.experimental.pallas import tpu as pltpu
from jax.experimental.pallas import tpu_sc as plsc

N = 10000
E = 320000
C_IN = 128
C_HID = 128
C_OUT = 64

NC = 2
NS = 16
NW = NC * NS
K = 128
CHUNKS = -(-E // (NW * K))
EP = NW * CHUNKS * K
NPAD = 10112
RPT = NPAD // NS

MB = 1000
GRID = N // MB


def _sc_mesh():
    return plsc.VectorSubcoreMesh(core_axis_name="c", subcore_axis_name="s")


def _deg_body(dst_hbm, ones_hbm, zeros_hbm, out_hbm, dstv, onesv, deg_sh):
    c = lax.axis_index("c")
    s = lax.axis_index("s")
    wid = c * NS + s
    pltpu.sync_copy(dst_hbm.at[wid], dstv)
    pltpu.sync_copy(ones_hbm, onesv)
    pltpu.sync_copy(zeros_hbm.at[pl.ds(s * RPT, RPT)],
                    deg_sh.at[pl.ds(s * RPT, RPT)])
    plsc.subcore_barrier()

    def body(j, carry):
        pltpu.sync_copy(onesv, deg_sh.at[dstv.at[j]], add=True)
        return carry

    lax.fori_loop(0, CHUNKS, body, 0)
    plsc.subcore_barrier()
    pltpu.sync_copy(deg_sh.at[pl.ds(s * RPT, RPT)],
                    out_hbm.at[c, pl.ds(s * RPT, RPT)])


@functools.lru_cache(maxsize=None)
def _make_deg(interpret=False):
    return pl.kernel(
        _deg_body,
        out_type=jax.ShapeDtypeStruct((NC, NPAD, C_IN), jnp.float32),
        mesh=_sc_mesh(),
        scratch_types=[
            pltpu.VMEM((CHUNKS, K), jnp.int32),
            pltpu.VMEM((K, C_IN), jnp.float32),
            pltpu.VMEM_SHARED((NPAD, C_IN), jnp.float32),
        ],
        interpret=interpret,
    )


def _agg_body(y_hbm, src_hbm, dst_hbm, zeros_hbm, out_hbm,
              srcv, dstv, rows, acc_sh, sem):
        c = lax.axis_index("c")
        s = lax.axis_index("s")
        wid = c * NS + s
        pltpu.sync_copy(src_hbm.at[wid], srcv)
        pltpu.sync_copy(dst_hbm.at[wid], dstv)
        pltpu.sync_copy(zeros_hbm.at[pl.ds(s * RPT, RPT)],
                        acc_sh.at[pl.ds(s * RPT, RPT)])
        plsc.subcore_barrier()

        def body(j, carry):
            pltpu.async_copy(y_hbm.at[srcv.at[j]], rows, sem).wait()
            pltpu.sync_copy(rows, acc_sh.at[dstv.at[j]], add=True)
            return carry

        lax.fori_loop(0, CHUNKS, body, 0)
        plsc.subcore_barrier()
        pltpu.sync_copy(acc_sh.at[pl.ds(s * RPT, RPT)],
                        out_hbm.at[c, pl.ds(s * RPT, RPT)])


@functools.lru_cache(maxsize=None)
def _make_agg(width, interpret=False):
    return pl.kernel(
        _agg_body,
        out_type=jax.ShapeDtypeStruct((NC, NPAD, width), jnp.float32),
        mesh=_sc_mesh(),
        scratch_types=[
            pltpu.VMEM((CHUNKS, K), jnp.int32),
            pltpu.VMEM((CHUNKS, K), jnp.int32),
            pltpu.VMEM((K, width), jnp.float32),
            pltpu.VMEM_SHARED((NPAD, width), jnp.float32),
            pltpu.SemaphoreType.DMA,
        ],
        interpret=interpret,
    )


def _tc1_body(degp_ref, x_ref, w1_ref, y1_ref, disb_ref):
    deg = degp_ref[0, :, 0:1] + degp_ref[1, :, 0:1]
    dis = lax.rsqrt(deg + 1.0)
    xw = jnp.dot(x_ref[...], w1_ref[...], preferred_element_type=jnp.float32)
    y1_ref[...] = xw * dis
    disb_ref[...] = jnp.broadcast_to(dis, (MB, C_IN))


def _tc2_body(p_ref, y1_ref, disb_ref, b1_ref, w2_ref, h_ref, y2_ref):
    acc = p_ref[0] + p_ref[1] + y1_ref[...]
    h = jnp.maximum(disb_ref[...] * acc + b1_ref[...], 0.0)
    h_ref[...] = h
    xw2 = jnp.dot(h, w2_ref[...], preferred_element_type=jnp.float32)
    y2_ref[...] = jnp.concatenate(
        [xw2 * disb_ref[:, :C_OUT], jnp.zeros((MB, C_HID - C_OUT), jnp.float32)],
        axis=-1)


def _tc3_body(p_ref, y2_ref, disb_ref, b2_ref, x_ref, h_ref,
              wl1_ref, wl2_ref, wl3_ref, bl_ref, out_ref):
    dis64 = disb_ref[:, :C_OUT]
    o = dis64 * (p_ref[0, :, :C_OUT] + p_ref[1, :, :C_OUT]
                 + y2_ref[:, :C_OUT]) + b2_ref[...]
    logits = (jnp.dot(x_ref[...], wl1_ref[...],
                      preferred_element_type=jnp.float32)
              + jnp.dot(h_ref[...], wl2_ref[...],
                        preferred_element_type=jnp.float32)
              + jnp.dot(o, wl3_ref[...], preferred_element_type=jnp.float32)
              + bl_ref[...])
    m = jnp.max(logits, axis=-1, keepdims=True)
    ls = logits - m
    out_ref[...] = ls - jnp.log(jnp.sum(jnp.exp(ls), axis=-1, keepdims=True))


_tc1 = pl.pallas_call(
    _tc1_body,
    grid=(GRID,),
    in_specs=[
        pl.BlockSpec((NC, MB, C_IN), lambda i: (0, i, 0)),
        pl.BlockSpec((MB, C_IN), lambda i: (i, 0)),
        pl.BlockSpec((C_IN, C_HID), lambda i: (0, 0)),
    ],
    out_specs=[
        pl.BlockSpec((MB, C_HID), lambda i: (i, 0)),
        pl.BlockSpec((MB, C_IN), lambda i: (i, 0)),
    ],
    out_shape=[
        jax.ShapeDtypeStruct((N, C_HID), jnp.float32),
        jax.ShapeDtypeStruct((N, C_IN), jnp.float32),
    ],
)

_tc2 = pl.pallas_call(
    _tc2_body,
    grid=(GRID,),
    in_specs=[
        pl.BlockSpec((NC, MB, C_HID), lambda i: (0, i, 0)),
        pl.BlockSpec((MB, C_HID), lambda i: (i, 0)),
        pl.BlockSpec((MB, C_IN), lambda i: (i, 0)),
        pl.BlockSpec((1, C_HID), lambda i: (0, 0)),
        pl.BlockSpec((C_HID, C_OUT), lambda i: (0, 0)),
    ],
    out_specs=[
        pl.BlockSpec((MB, C_HID), lambda i: (i, 0)),
        pl.BlockSpec((MB, C_HID), lambda i: (i, 0)),
    ],
    out_shape=[
        jax.ShapeDtypeStruct((N, C_HID), jnp.float32),
        jax.ShapeDtypeStruct((N, C_HID), jnp.float32),
    ],
)

_tc3 = pl.pallas_call(
    _tc3_body,
    grid=(GRID,),
    in_specs=[
        pl.BlockSpec((NC, MB, C_HID), lambda i: (0, i, 0)),
        pl.BlockSpec((MB, C_HID), lambda i: (i, 0)),
        pl.BlockSpec((MB, C_IN), lambda i: (i, 0)),
        pl.BlockSpec((1, C_OUT), lambda i: (0, 0)),
        pl.BlockSpec((MB, C_IN), lambda i: (i, 0)),
        pl.BlockSpec((MB, C_HID), lambda i: (i, 0)),
        pl.BlockSpec((C_IN, C_OUT), lambda i: (0, 0)),
        pl.BlockSpec((C_HID, C_OUT), lambda i: (0, 0)),
        pl.BlockSpec((C_OUT, C_OUT), lambda i: (0, 0)),
        pl.BlockSpec((1, C_OUT), lambda i: (0, 0)),
    ],
    out_specs=pl.BlockSpec((MB, C_OUT), lambda i: (i, 0)),
    out_shape=jax.ShapeDtypeStruct((N, C_OUT), jnp.float32),
)


def kernel(x, edge_index, W1, b1, W2, b2, Wl, bl):
    ei = edge_index.astype(jnp.int32)
    src, dst = ei[0], ei[1]
    srcp = jnp.concatenate(
        [src, jnp.zeros((EP - E,), jnp.int32)]).reshape(NW, CHUNKS, K)
    pad_dst = N + (jnp.arange(EP - E, dtype=jnp.int32) % (NPAD - N))
    dstp = jnp.concatenate([dst, pad_dst]).reshape(NW, CHUNKS, K)

    ones128 = jnp.ones((K, C_IN), jnp.float32)
    z128 = jnp.zeros((NPAD, C_HID), jnp.float32)

    degp = _make_deg()(dstp, ones128, z128)
    y1, disb = _tc1(degp, x, W1)
    agg = _make_agg(C_HID)
    p1 = agg(y1, srcp, dstp, z128)
    h, y2 = _tc2(p1, y1, disb, b1.reshape(1, -1), W2)
    p2 = agg(y2, srcp, dstp, z128)
    out = _tc3(p2, y2, disb, b2.reshape(1, -1), x, h,
               Wl[:C_IN], Wl[C_IN:C_IN + C_HID], Wl[C_IN + C_HID:],
               bl.reshape(1, -1))
    return out


# speedup vs baseline: 1.0437x; 1.0437x over previous
import functools

import jax
import jax.numpy as jnp
from jax import lax
from jax.experimental import pallas as pl
from jax.experimental.pallas import tpu as pltpu
from jax.experimental.pallas import tpu_sc as plsc

N = 10000
E = 320000
C_IN = 128
C_HID = 128
C_OUT = 64

NC = 2
NS = 16
NW = NC * NS
K = 128
CHUNKS = -(-E // (NW * K))
EP = NW * CHUNKS * K
NPAD = 10112
RPT = NPAD // NS

MB = 1000
GRID = N // MB


def _sc_mesh():
    return plsc.VectorSubcoreMesh(core_axis_name="c", subcore_axis_name="s")


def _deg_body(dst_hbm, ones_hbm, zeros_hbm, out_hbm, dstv, onesv, deg_sh):
    c = lax.axis_index("c")
    s = lax.axis_index("s")
    wid = c * NS + s
    pltpu.sync_copy(dst_hbm.at[wid], dstv)
    pltpu.sync_copy(ones_hbm, onesv)
    pltpu.sync_copy(zeros_hbm.at[pl.ds(s * RPT, RPT)],
                    deg_sh.at[pl.ds(s * RPT, RPT)])
    plsc.subcore_barrier()

    def body(j, carry):
        pltpu.sync_copy(onesv, deg_sh.at[dstv.at[j]], add=True)
        return carry

    lax.fori_loop(0, CHUNKS, body, 0)
    plsc.subcore_barrier()
    pltpu.sync_copy(deg_sh.at[pl.ds(s * RPT, RPT)],
                    out_hbm.at[c, pl.ds(s * RPT, RPT)])


@functools.lru_cache(maxsize=None)
def _make_deg(interpret=False):
    return pl.kernel(
        _deg_body,
        out_type=jax.ShapeDtypeStruct((NC, NPAD, C_IN), jnp.float32),
        mesh=_sc_mesh(),
        scratch_types=[
            pltpu.VMEM((CHUNKS, K), jnp.int32),
            pltpu.VMEM((K, C_IN), jnp.float32),
            pltpu.VMEM_SHARED((NPAD, C_IN), jnp.float32),
        ],
        interpret=interpret,
    )


def _agg_body(y_hbm, src_hbm, dst_hbm, zeros_hbm, out_hbm,
              srcv, dstv, rows, acc_sh, sem):
        c = lax.axis_index("c")
        s = lax.axis_index("s")
        wid = c * NS + s
        pltpu.sync_copy(src_hbm.at[wid], srcv)
        pltpu.sync_copy(dst_hbm.at[wid], dstv)
        pltpu.sync_copy(zeros_hbm.at[pl.ds(s * RPT, RPT)],
                        acc_sh.at[pl.ds(s * RPT, RPT)])
        plsc.subcore_barrier()

        def body(j, carry):
            pltpu.async_copy(y_hbm.at[srcv.at[j]], rows, sem).wait()
            pltpu.sync_copy(rows, acc_sh.at[dstv.at[j]], add=True)
            return carry

        lax.fori_loop(0, CHUNKS, body, 0)
        plsc.subcore_barrier()
        pltpu.sync_copy(acc_sh.at[pl.ds(s * RPT, RPT)],
                        out_hbm.at[c, pl.ds(s * RPT, RPT)])


@functools.lru_cache(maxsize=None)
def _make_agg(width, interpret=False):
    return pl.kernel(
        _agg_body,
        out_type=jax.ShapeDtypeStruct((NC, NPAD, width), jnp.float32),
        mesh=_sc_mesh(),
        scratch_types=[
            pltpu.VMEM((CHUNKS, K), jnp.int32),
            pltpu.VMEM((CHUNKS, K), jnp.int32),
            pltpu.VMEM((K, width), jnp.float32),
            pltpu.VMEM_SHARED((NPAD, width), jnp.float32),
            pltpu.SemaphoreType.DMA,
        ],
        interpret=interpret,
    )


def _tc1_body(degp_ref, x_ref, w1_ref, y1_ref, disb_ref):
    deg = degp_ref[0, :, 0:1] + degp_ref[1, :, 0:1]
    dis = lax.rsqrt(deg + 1.0)
    xw = jnp.dot(x_ref[...], w1_ref[...], preferred_element_type=jnp.float32)
    y1_ref[...] = xw * dis
    disb_ref[...] = jnp.broadcast_to(dis, (MB, C_IN))


def _tc2_body(p_ref, y1_ref, disb_ref, b1_ref, w2_ref, h_ref, y2_ref):
    acc = p_ref[0] + p_ref[1] + y1_ref[...]
    h = jnp.maximum(disb_ref[...] * acc + b1_ref[...], 0.0)
    h_ref[...] = h
    xw2 = jnp.dot(h, w2_ref[...], preferred_element_type=jnp.float32)
    y2_ref[...] = jnp.concatenate(
        [xw2 * disb_ref[:, :C_OUT], jnp.zeros((MB, C_HID - C_OUT), jnp.float32)],
        axis=-1)


def _tc3_body(p_ref, y2_ref, disb_ref, b2_ref, x_ref, h_ref,
              wl1_ref, wl2_ref, wl3_ref, bl_ref, out_ref):
    dis64 = disb_ref[:, :C_OUT]
    o = dis64 * (p_ref[0] + p_ref[1] + y2_ref[:, :C_OUT]) + b2_ref[...]
    logits = (jnp.dot(x_ref[...], wl1_ref[...],
                      preferred_element_type=jnp.float32)
              + jnp.dot(h_ref[...], wl2_ref[...],
                        preferred_element_type=jnp.float32)
              + jnp.dot(o, wl3_ref[...], preferred_element_type=jnp.float32)
              + bl_ref[...])
    m = jnp.max(logits, axis=-1, keepdims=True)
    ls = logits - m
    out_ref[...] = ls - jnp.log(jnp.sum(jnp.exp(ls), axis=-1, keepdims=True))


_tc1 = pl.pallas_call(
    _tc1_body,
    grid=(GRID,),
    in_specs=[
        pl.BlockSpec((NC, MB, C_IN), lambda i: (0, i, 0)),
        pl.BlockSpec((MB, C_IN), lambda i: (i, 0)),
        pl.BlockSpec((C_IN, C_HID), lambda i: (0, 0)),
    ],
    out_specs=[
        pl.BlockSpec((MB, C_HID), lambda i: (i, 0)),
        pl.BlockSpec((MB, C_IN), lambda i: (i, 0)),
    ],
    out_shape=[
        jax.ShapeDtypeStruct((N, C_HID), jnp.float32),
        jax.ShapeDtypeStruct((N, C_IN), jnp.float32),
    ],
)

_tc2 = pl.pallas_call(
    _tc2_body,
    grid=(GRID,),
    in_specs=[
        pl.BlockSpec((NC, MB, C_HID), lambda i: (0, i, 0)),
        pl.BlockSpec((MB, C_HID), lambda i: (i, 0)),
        pl.BlockSpec((MB, C_IN), lambda i: (i, 0)),
        pl.BlockSpec((1, C_HID), lambda i: (0, 0)),
        pl.BlockSpec((C_HID, C_OUT), lambda i: (0, 0)),
    ],
    out_specs=[
        pl.BlockSpec((MB, C_HID), lambda i: (i, 0)),
        pl.BlockSpec((MB, C_HID), lambda i: (i, 0)),
    ],
    out_shape=[
        jax.ShapeDtypeStruct((N, C_HID), jnp.float32),
        jax.ShapeDtypeStruct((N, C_HID), jnp.float32),
    ],
)

_tc3 = pl.pallas_call(
    _tc3_body,
    grid=(GRID,),
    in_specs=[
        pl.BlockSpec((NC, MB, C_OUT), lambda i: (0, i, 0)),
        pl.BlockSpec((MB, C_HID), lambda i: (i, 0)),
        pl.BlockSpec((MB, C_IN), lambda i: (i, 0)),
        pl.BlockSpec((1, C_OUT), lambda i: (0, 0)),
        pl.BlockSpec((MB, C_IN), lambda i: (i, 0)),
        pl.BlockSpec((MB, C_HID), lambda i: (i, 0)),
        pl.BlockSpec((C_IN, C_OUT), lambda i: (0, 0)),
        pl.BlockSpec((C_HID, C_OUT), lambda i: (0, 0)),
        pl.BlockSpec((C_OUT, C_OUT), lambda i: (0, 0)),
        pl.BlockSpec((1, C_OUT), lambda i: (0, 0)),
    ],
    out_specs=pl.BlockSpec((MB, C_OUT), lambda i: (i, 0)),
    out_shape=jax.ShapeDtypeStruct((N, C_OUT), jnp.float32),
)


def kernel(x, edge_index, W1, b1, W2, b2, Wl, bl):
    ei = edge_index.astype(jnp.int32)
    src, dst = ei[0], ei[1]
    srcp = jnp.concatenate(
        [src, jnp.zeros((EP - E,), jnp.int32)]).reshape(NW, CHUNKS, K)
    pad_dst = N + (jnp.arange(EP - E, dtype=jnp.int32) % (NPAD - N))
    dstp = jnp.concatenate([dst, pad_dst]).reshape(NW, CHUNKS, K)

    ones128 = jnp.ones((K, C_IN), jnp.float32)
    z128 = jnp.zeros((NPAD, C_HID), jnp.float32)

    degp = _make_deg()(dstp, ones128, z128)[:, :N, :]
    y1, disb = _tc1(degp, x, W1)
    agg = _make_agg(C_HID)
    p1 = agg(y1, srcp, dstp, z128)[:, :N, :]
    h, y2 = _tc2(p1, y1, disb, b1.reshape(1, -1), W2)
    p2 = agg(y2, srcp, dstp, z128)[:, :N, :C_OUT]
    out = _tc3(p2, y2, disb, b2.reshape(1, -1), x, h,
               Wl[:C_IN], Wl[C_IN:C_IN + C_HID], Wl[C_IN + C_HID:],
               bl.reshape(1, -1))
    return out
